# Initial kernel scaffold; baseline (speedup 1.0000x reference)
import jax
import jax.numpy as jnp
from jax.experimental import pallas as pl


def kernel(predicted_locs, predicted_scores, boxes, labels, priors_cxcy):
    raise NotImplementedError("write your pallas kernel here")


# speedup vs baseline: 6.9159x; 6.9159x over previous
import functools

import jax
import jax.numpy as jnp
from jax import lax
from jax.experimental import pallas as pl
from jax.experimental.pallas import tpu as pltpu

_THRESHOLD = 0.5
_NEG_POS_RATIO = 3.0
_ALPHA = 1.0

_P = 8732
_PPAD = 8832
_ROWS = 69
_LANES = 128
_O = 12
_C = 21


def _match_body(boxes_ref, priors_ref, label_ref, tl_ref):
    px0 = priors_ref[0]
    py0 = priors_ref[1]
    px1 = priors_ref[2]
    py1 = priors_ref[3]
    parea = priors_ref[10]

    pidx = lax.broadcasted_iota(jnp.int32, (_ROWS, _LANES), 0) * _LANES + \
        lax.broadcasted_iota(jnp.int32, (_ROWS, _LANES), 1)

    best_iou = jnp.zeros((_ROWS, _LANES), jnp.float32)
    best_obj = jnp.zeros((_ROWS, _LANES), jnp.int32)
    prior_for_obj = []
    for o in range(_O):
        bx0 = boxes_ref[0, o, 0]
        by0 = boxes_ref[0, o, 1]
        bx1 = boxes_ref[0, o, 2]
        by1 = boxes_ref[0, o, 3]
        barea = boxes_ref[0, o, 8]
        lox = jnp.maximum(px0, bx0)
        loy = jnp.maximum(py0, by0)
        hix = jnp.minimum(px1, bx1)
        hiy = jnp.minimum(py1, by1)
        inter = jnp.maximum(hix - lox, 0.0) * jnp.maximum(hiy - loy, 0.0)
        iou = inter / (parea + barea - inter)
        m = jnp.max(iou)
        idx = jnp.min(jnp.where(iou == m, pidx, jnp.int32(2**30)))
        prior_for_obj.append(idx)
        upd = iou > best_iou
        best_iou = jnp.where(upd, iou, best_iou)
        best_obj = jnp.where(upd, o, best_obj)

    for o in range(_O):
        hit = pidx == prior_for_obj[o]
        best_obj = jnp.where(hit, o, best_obj)
        best_iou = jnp.where(hit, 1.0, best_iou)

    lab = jnp.zeros((_ROWS, _LANES), jnp.float32)
    gcx = jnp.zeros((_ROWS, _LANES), jnp.float32)
    gcy = jnp.zeros((_ROWS, _LANES), jnp.float32)
    gw = jnp.zeros((_ROWS, _LANES), jnp.float32)
    gh = jnp.zeros((_ROWS, _LANES), jnp.float32)
    for o in range(_O):
        sel = best_obj == o
        lab = jnp.where(sel, boxes_ref[0, o, 9], lab)
        gcx = jnp.where(sel, boxes_ref[0, o, 4], gcx)
        gcy = jnp.where(sel, boxes_ref[0, o, 5], gcy)
        gw = jnp.where(sel, boxes_ref[0, o, 6], gw)
        gh = jnp.where(sel, boxes_ref[0, o, 7], gh)

    lab_i = lab.astype(jnp.int32)
    lab_i = jnp.where(best_iou < _THRESHOLD, 0, lab_i)
    label_ref[0] = lab_i

    pcx = priors_ref[4]
    pcy = priors_ref[5]
    i10w = priors_ref[6]
    i10h = priors_ref[7]
    iw = priors_ref[8]
    ih = priors_ref[9]
    tl_ref[0, 0] = (gcx - pcx) * i10w
    tl_ref[0, 1] = (gcy - pcy) * i10h
    tl_ref[0, 2] = jnp.log(gw * iw) * 5.0
    tl_ref[0, 3] = jnp.log(gh * ih) * 5.0


def _ce_body(scores_ref, lbl_ref, ce_ref, stats_ref):
    s = scores_ref[0]
    lbl = lbl_ref[0]
    ones = jnp.ones((_C, 1), jnp.float32)
    e = jnp.exp(s)
    denom = lax.dot_general(e, ones, (((1,), (0,)), ((), ())),
                            preferred_element_type=jnp.float32)
    onehot = (lax.broadcasted_iota(jnp.int32, (_P, _C), 1) == lbl)
    st = lax.dot_general(jnp.where(onehot, s, 0.0), ones,
                         (((1,), (0,)), ((), ())),
                         preferred_element_type=jnp.float32)
    ce = jnp.log(denom) - st
    ce_ref[0] = ce
    posf = (lbl != 0).astype(jnp.float32)
    npos = jnp.sum(posf)
    cpos = jnp.sum(ce * posf)
    li = lax.broadcasted_iota(jnp.int32, (1, _LANES), 1)
    stats_ref[0] = jnp.where(li == 0, npos, jnp.where(li == 1, cpos, 0.0))


def _mine_body(ce_ref, label_ref, locs_ref, tl_ref, stats_ref, out_ref,
               acc_ref):
    i = pl.program_id(0)

    @pl.when(i == 0)
    def _():
        acc_ref[0] = 0.0
        acc_ref[1] = 0.0
        acc_ref[2] = 0.0
        acc_ref[3] = 0.0

    lab = label_ref[0]
    pos = lab != 0
    posf = pos.astype(jnp.float32)
    ce = ce_ref[0]
    ce_neg = jnp.where(pos, 0.0, ce)
    ceb = lax.bitcast_convert_type(ce_neg, jnp.int32)

    npos = stats_ref[0, 0, 0]
    cpos = stats_ref[0, 0, 1]
    k = (npos * _NEG_POS_RATIO).astype(jnp.int32)

    lo = jnp.int32(0)
    for bit in range(30, -1, -1):
        cand = lo | jnp.int32(1 << bit)
        cnt = jnp.sum((ceb >= cand).astype(jnp.int32))
        lo = jnp.where(cnt >= k, cand, lo)

    gt = ceb > lo
    hard_sum = jnp.sum(jnp.where(gt, ce_neg, 0.0))
    cnt_gt = jnp.sum(gt.astype(jnp.int32))
    t = lax.bitcast_convert_type(jnp.full((8, _LANES), lo, jnp.int32),
                                 jnp.float32)[0, 0]
    hard = jnp.where(k > 0, hard_sum + t * (k - cnt_gt).astype(jnp.float32),
                     0.0)

    labs = jnp.float32(0.0)
    for j in range(4):
        labs += jnp.sum(jnp.abs(locs_ref[0, j] - tl_ref[0, j]) * posf)

    acc_ref[0] += hard
    acc_ref[1] += cpos
    acc_ref[2] += npos
    acc_ref[3] += labs

    @pl.when(i == pl.num_programs(0) - 1)
    def _():
        npt = acc_ref[2]
        loss = (acc_ref[0] + acc_ref[1]) / npt + \
            _ALPHA * acc_ref[3] / (npt * 4.0)
        out_ref[...] = jnp.full((1, _LANES), loss, jnp.float32)


@jax.jit
def _run(predicted_locs, predicted_scores, boxes, labels, priors_cxcy):
    B = predicted_scores.shape[0]

    pxy0 = priors_cxcy[:, :2] - priors_cxcy[:, 2:] / 2.0
    pxy1 = priors_cxcy[:, :2] + priors_cxcy[:, 2:] / 2.0
    parea = (pxy1[:, 0] - pxy0[:, 0]) * (pxy1[:, 1] - pxy0[:, 1])
    cols = [pxy0[:, 0], pxy0[:, 1], pxy1[:, 0], pxy1[:, 1],
            priors_cxcy[:, 0], priors_cxcy[:, 1],
            10.0 / priors_cxcy[:, 2], 10.0 / priors_cxcy[:, 3],
            1.0 / priors_cxcy[:, 2], 1.0 / priors_cxcy[:, 3], parea]
    pstack = jnp.stack(cols, 0)
    pad_vals = jnp.array([-5.5, -5.5, -4.5, -4.5, -5.0, -5.0,
                          10.0, 10.0, 1.0, 1.0, 1.0], jnp.float32)
    pad_blk = jnp.broadcast_to(pad_vals[:, None], (11, _PPAD - _P))
    pstack = jnp.concatenate([pstack, pad_blk], 1).reshape(11, _ROWS, _LANES)

    bxy0 = boxes[..., :2]
    bxy1 = boxes[..., 2:]
    bwh = bxy1 - bxy0
    bcxy = (bxy0 + bxy1) / 2.0
    barea = (bwh[..., 0] * bwh[..., 1])[..., None]
    boxes_aug = jnp.concatenate(
        [bxy0, bxy1, bcxy, bwh, barea, labels[..., None].astype(jnp.float32)],
        -1)

    label_pl, tl_pl = pl.pallas_call(
        _match_body,
        grid=(B,),
        in_specs=[
            pl.BlockSpec((1, _O, 10), lambda i: (i, 0, 0)),
            pl.BlockSpec((11, _ROWS, _LANES), lambda i: (0, 0, 0)),
        ],
        out_specs=[
            pl.BlockSpec((1, _ROWS, _LANES), lambda i: (i, 0, 0)),
            pl.BlockSpec((1, 4, _ROWS, _LANES), lambda i: (i, 0, 0, 0)),
        ],
        out_shape=[
            jax.ShapeDtypeStruct((B, _ROWS, _LANES), jnp.int32),
            jax.ShapeDtypeStruct((B, 4, _ROWS, _LANES), jnp.float32),
        ],
    )(boxes_aug, pstack)

    lbl_col = label_pl.reshape(B, _PPAD)[:, :_P].reshape(B, _P, 1)
    ce_col, stats = pl.pallas_call(
        _ce_body,
        grid=(B,),
        in_specs=[
            pl.BlockSpec((1, _P, _C), lambda i: (i, 0, 0)),
            pl.BlockSpec((1, _P, 1), lambda i: (i, 0, 0)),
        ],
        out_specs=[
            pl.BlockSpec((1, _P, 1), lambda i: (i, 0, 0)),
            pl.BlockSpec((1, 1, _LANES), lambda i: (i, 0, 0)),
        ],
        out_shape=[
            jax.ShapeDtypeStruct((B, _P, 1), jnp.float32),
            jax.ShapeDtypeStruct((B, 1, _LANES), jnp.float32),
        ],
    )(predicted_scores, lbl_col)

    ce_planes = jnp.pad(ce_col.reshape(B, _P), ((0, 0), (0, _PPAD - _P)))
    ce_planes = ce_planes.reshape(B, _ROWS, _LANES)
    locs_t = jnp.moveaxis(predicted_locs, 2, 1)
    locs_t = jnp.pad(locs_t, ((0, 0), (0, 0), (0, _PPAD - _P)))
    locs_t = locs_t.reshape(B, 4, _ROWS, _LANES)

    out = pl.pallas_call(
        _mine_body,
        grid=(B,),
        in_specs=[
            pl.BlockSpec((1, _ROWS, _LANES), lambda i: (i, 0, 0)),
            pl.BlockSpec((1, _ROWS, _LANES), lambda i: (i, 0, 0)),
            pl.BlockSpec((1, 4, _ROWS, _LANES), lambda i: (i, 0, 0, 0)),
            pl.BlockSpec((1, 4, _ROWS, _LANES), lambda i: (i, 0, 0, 0)),
            pl.BlockSpec((1, 1, _LANES), lambda i: (i, 0, 0)),
        ],
        out_specs=pl.BlockSpec((1, _LANES), lambda i: (0, 0)),
        out_shape=jax.ShapeDtypeStruct((1, _LANES), jnp.float32),
        scratch_shapes=[pltpu.SMEM((4,), jnp.float32)],
    )(ce_planes, label_pl, locs_t, tl_pl, stats)

    return out[0, 0]


def kernel(predicted_locs, predicted_scores, boxes, labels, priors_cxcy):
    return _run(predicted_locs, predicted_scores, boxes, labels, priors_cxcy)


# speedup vs baseline: 8.7766x; 1.2690x over previous
import functools

import jax
import jax.numpy as jnp
from jax import lax
from jax.experimental import pallas as pl
from jax.experimental.pallas import tpu as pltpu

_THRESHOLD = 0.5
_NEG_POS_RATIO = 3.0
_ALPHA = 1.0

_P = 8732
_PPAD = 8832
_ROWS = 69
_LANES = 128
_O = 12
_C = 21


def _match_body(boxes_ref, priors_ref, label_ref, tl_ref):
    px0 = priors_ref[0]
    py0 = priors_ref[1]
    px1 = priors_ref[2]
    py1 = priors_ref[3]
    parea = priors_ref[10]

    pidx = lax.broadcasted_iota(jnp.int32, (_ROWS, _LANES), 0) * _LANES + \
        lax.broadcasted_iota(jnp.int32, (_ROWS, _LANES), 1)

    best_iou = jnp.zeros((_ROWS, _LANES), jnp.float32)
    best_obj = jnp.zeros((_ROWS, _LANES), jnp.int32)
    prior_for_obj = []
    for o in range(_O):
        bx0 = boxes_ref[0, o, 0]
        by0 = boxes_ref[0, o, 1]
        bx1 = boxes_ref[0, o, 2]
        by1 = boxes_ref[0, o, 3]
        barea = boxes_ref[0, o, 8]
        lox = jnp.maximum(px0, bx0)
        loy = jnp.maximum(py0, by0)
        hix = jnp.minimum(px1, bx1)
        hiy = jnp.minimum(py1, by1)
        inter = jnp.maximum(hix - lox, 0.0) * jnp.maximum(hiy - loy, 0.0)
        iou = inter / (parea + barea - inter)
        m = jnp.max(iou)
        idx = jnp.min(jnp.where(iou == m, pidx, jnp.int32(2**30)))
        prior_for_obj.append(idx)
        upd = iou > best_iou
        best_iou = jnp.where(upd, iou, best_iou)
        best_obj = jnp.where(upd, o, best_obj)

    for o in range(_O):
        hit = pidx == prior_for_obj[o]
        best_obj = jnp.where(hit, o, best_obj)
        best_iou = jnp.where(hit, 1.0, best_iou)

    lab = jnp.zeros((_ROWS, _LANES), jnp.float32)
    gcx = jnp.zeros((_ROWS, _LANES), jnp.float32)
    gcy = jnp.zeros((_ROWS, _LANES), jnp.float32)
    gw = jnp.zeros((_ROWS, _LANES), jnp.float32)
    gh = jnp.zeros((_ROWS, _LANES), jnp.float32)
    for o in range(_O):
        sel = best_obj == o
        lab = jnp.where(sel, boxes_ref[0, o, 9], lab)
        gcx = jnp.where(sel, boxes_ref[0, o, 4], gcx)
        gcy = jnp.where(sel, boxes_ref[0, o, 5], gcy)
        gw = jnp.where(sel, boxes_ref[0, o, 6], gw)
        gh = jnp.where(sel, boxes_ref[0, o, 7], gh)

    lab_i = lab.astype(jnp.int32)
    lab_i = jnp.where(best_iou < _THRESHOLD, 0, lab_i)
    label_ref[0] = lab_i

    pcx = priors_ref[4]
    pcy = priors_ref[5]
    i10w = priors_ref[6]
    i10h = priors_ref[7]
    iw = priors_ref[8]
    ih = priors_ref[9]
    tl_ref[0, 0] = (gcx - pcx) * i10w
    tl_ref[0, 1] = (gcy - pcy) * i10h
    tl_ref[0, 2] = jnp.log(gw * iw) * 5.0
    tl_ref[0, 3] = jnp.log(gh * ih) * 5.0


def _ce_body(scores_ref, lbl_ref, ce_ref):
    s = scores_ref[0]
    lbl = lbl_ref[0]
    ones = jnp.ones((_C, 1), jnp.float32)
    e = jnp.exp(s)
    denom = lax.dot_general(e, ones, (((1,), (0,)), ((), ())),
                            preferred_element_type=jnp.float32)
    onehot = (lax.broadcasted_iota(jnp.int32, (_P, _C), 1) == lbl)
    st = lax.dot_general(jnp.where(onehot, s, 0.0), ones,
                         (((1,), (0,)), ((), ())),
                         preferred_element_type=jnp.float32)
    ce_ref[0] = jnp.log(denom) - st


def _mine_body(ce_ref, label_ref, locs_ref, tl_ref, out_ref):
    lab = label_ref[...]
    pos = lab != 0
    posf = pos.astype(jnp.float32)
    ce = ce_ref[...]
    ce_neg = jnp.where(pos, 0.0, ce)
    ceb = lax.bitcast_convert_type(ce_neg, jnp.int32)

    npos = jnp.sum(posf, axis=1, keepdims=True)
    cpos = jnp.sum(ce * posf)
    k = (npos * _NEG_POS_RATIO).astype(jnp.int32)

    lo = jnp.zeros_like(k)
    for bit in range(30, -1, -1):
        cand = lo | jnp.int32(1 << bit)
        cnt = jnp.sum((ceb >= cand).astype(jnp.int32), axis=1, keepdims=True)
        lo = jnp.where(cnt >= k, cand, lo)

    gt = ceb > lo
    hard_sum = jnp.sum(jnp.where(gt, ce_neg, 0.0), axis=1, keepdims=True)
    cnt_gt = jnp.sum(gt.astype(jnp.int32), axis=1, keepdims=True)
    t = lax.bitcast_convert_type(lo, jnp.float32)
    hard_row = jnp.where(k > 0,
                         hard_sum + t * (k - cnt_gt).astype(jnp.float32),
                         0.0)
    hard = jnp.sum(hard_row)

    labs = jnp.sum(jnp.abs(locs_ref[...] - tl_ref[...]) * posf[:, None, :])

    npt = jnp.sum(npos)
    loss = (hard + cpos) / npt + _ALPHA * labs / (npt * 4.0)
    out_ref[...] = jnp.full((1, _LANES), loss, jnp.float32)


@jax.jit
def _run(predicted_locs, predicted_scores, boxes, labels, priors_cxcy):
    B = predicted_scores.shape[0]

    pxy0 = priors_cxcy[:, :2] - priors_cxcy[:, 2:] / 2.0
    pxy1 = priors_cxcy[:, :2] + priors_cxcy[:, 2:] / 2.0
    parea = (pxy1[:, 0] - pxy0[:, 0]) * (pxy1[:, 1] - pxy0[:, 1])
    cols = [pxy0[:, 0], pxy0[:, 1], pxy1[:, 0], pxy1[:, 1],
            priors_cxcy[:, 0], priors_cxcy[:, 1],
            10.0 / priors_cxcy[:, 2], 10.0 / priors_cxcy[:, 3],
            1.0 / priors_cxcy[:, 2], 1.0 / priors_cxcy[:, 3], parea]
    pstack = jnp.stack(cols, 0)
    pad_vals = jnp.array([-5.5, -5.5, -4.5, -4.5, -5.0, -5.0,
                          10.0, 10.0, 1.0, 1.0, 1.0], jnp.float32)
    pad_blk = jnp.broadcast_to(pad_vals[:, None], (11, _PPAD - _P))
    pstack = jnp.concatenate([pstack, pad_blk], 1).reshape(11, _ROWS, _LANES)

    bxy0 = boxes[..., :2]
    bxy1 = boxes[..., 2:]
    bwh = bxy1 - bxy0
    bcxy = (bxy0 + bxy1) / 2.0
    barea = (bwh[..., 0] * bwh[..., 1])[..., None]
    boxes_aug = jnp.concatenate(
        [bxy0, bxy1, bcxy, bwh, barea, labels[..., None].astype(jnp.float32)],
        -1)

    label_pl, tl_pl = pl.pallas_call(
        _match_body,
        grid=(B,),
        in_specs=[
            pl.BlockSpec((1, _O, 10), lambda i: (i, 0, 0)),
            pl.BlockSpec((11, _ROWS, _LANES), lambda i: (0, 0, 0)),
        ],
        out_specs=[
            pl.BlockSpec((1, _ROWS, _LANES), lambda i: (i, 0, 0)),
            pl.BlockSpec((1, 4, _ROWS, _LANES), lambda i: (i, 0, 0, 0)),
        ],
        out_shape=[
            jax.ShapeDtypeStruct((B, _ROWS, _LANES), jnp.int32),
            jax.ShapeDtypeStruct((B, 4, _ROWS, _LANES), jnp.float32),
        ],
    )(boxes_aug, pstack)

    lbl_rows = label_pl.reshape(B, _PPAD)[:, :_P]
    lbl_col = lbl_rows.reshape(B, _P, 1)
    ce_col = pl.pallas_call(
        _ce_body,
        grid=(B,),
        in_specs=[
            pl.BlockSpec((1, _P, _C), lambda i: (i, 0, 0)),
            pl.BlockSpec((1, _P, 1), lambda i: (i, 0, 0)),
        ],
        out_specs=pl.BlockSpec((1, _P, 1), lambda i: (i, 0, 0)),
        out_shape=jax.ShapeDtypeStruct((B, _P, 1), jnp.float32),
    )(predicted_scores, lbl_col)

    ce_rows = ce_col.reshape(B, _P)
    locs_t = jnp.moveaxis(predicted_locs, 2, 1)
    tl_rows = tl_pl.reshape(B, 4, _PPAD)[:, :, :_P]

    out = pl.pallas_call(
        _mine_body,
        in_specs=[
            pl.BlockSpec((B, _P), lambda: (0, 0)),
            pl.BlockSpec((B, _P), lambda: (0, 0)),
            pl.BlockSpec((B, 4, _P), lambda: (0, 0, 0)),
            pl.BlockSpec((B, 4, _P), lambda: (0, 0, 0)),
        ],
        out_specs=pl.BlockSpec((1, _LANES), lambda: (0, 0)),
        out_shape=jax.ShapeDtypeStruct((1, _LANES), jnp.float32),
    )(ce_rows, lbl_rows, locs_t, tl_rows)

    return out[0, 0]


def kernel(predicted_locs, predicted_scores, boxes, labels, priors_cxcy):
    return _run(predicted_locs, predicted_scores, boxes, labels, priors_cxcy)


# speedup vs baseline: 17.1030x; 1.9487x over previous
import jax
import jax.numpy as jnp
from jax import lax
from jax.experimental import pallas as pl
from jax.experimental.pallas import tpu as pltpu

_THRESHOLD = 0.5
_NEG_POS_RATIO = 3.0
_ALPHA = 1.0

_P = 8732
_PPAD = 8832
_ROWS = 69
_LANES = 128
_O = 12
_C = 21
_B = 64
_BIG = 2**30


def _match_body(boxes_ref, priors_ref, locs_ref, label_ref, labs_ref,
                bi_ref, bo_ref):
    lane = lax.broadcasted_iota(jnp.int32, (_B, _LANES), 1)

    pp = jnp.zeros((_B, _LANES), jnp.int32)
    for o in range(_O):
        c0 = o * 10
        bx0 = boxes_ref[:, c0:c0 + 1]
        by0 = boxes_ref[:, c0 + 1:c0 + 2]
        bx1 = boxes_ref[:, c0 + 2:c0 + 3]
        by1 = boxes_ref[:, c0 + 3:c0 + 4]
        barea = boxes_ref[:, c0 + 8:c0 + 9]

        def p1_body(c, carry, o=o, bx0=bx0, by0=by0, bx1=bx1, by1=by1,
                    barea=barea):
            bl, ci = carry
            sl = pl.ds(c * _LANES, _LANES)
            inter = jnp.maximum(jnp.minimum(priors_ref[2, c], bx1) -
                                jnp.maximum(priors_ref[0, c], bx0), 0.0) * \
                jnp.maximum(jnp.minimum(priors_ref[3, c], by1) -
                            jnp.maximum(priors_ref[1, c], by0), 0.0)
            iou = inter / (priors_ref[10, c] + barea - inter)
            upd = iou > bl
            bl = jnp.where(upd, iou, bl)
            ci = jnp.where(upd, c, ci)
            if o == 0:
                bi_ref[:, sl] = iou
                bo_ref[:, sl] = jnp.zeros((_B, _LANES), jnp.int32)
            else:
                bic = bi_ref[:, sl]
                upd2 = iou > bic
                bi_ref[:, sl] = jnp.where(upd2, iou, bic)
                bo_ref[:, sl] = jnp.where(upd2, o, bo_ref[:, sl])
            return bl, ci

        bl, ci = lax.fori_loop(
            0, _ROWS, p1_body,
            (jnp.zeros((_B, _LANES), jnp.float32),
             jnp.zeros((_B, _LANES), jnp.int32)))
        m = jnp.max(bl, axis=1, keepdims=True)
        p_o = jnp.min(jnp.where(bl == m, ci * _LANES + lane, _BIG),
                      axis=1, keepdims=True)
        pp = jnp.where(lane == o, p_o, pp)

    def p2_body(c, labs_acc):
        sl = pl.ds(c * _LANES, _LANES)
        pvec = lane + c * _LANES
        bic = bi_ref[:, sl]
        boc = bo_ref[:, sl]
        for o in range(_O):
            hit = pvec == lax.slice(pp, (0, o), (_B, o + 1))
            boc = jnp.where(hit, o, boc)
            bic = jnp.where(hit, 1.0, bic)
        lab = jnp.zeros((_B, _LANES), jnp.float32)
        gcx = jnp.zeros((_B, _LANES), jnp.float32)
        gcy = jnp.zeros((_B, _LANES), jnp.float32)
        gw = jnp.zeros((_B, _LANES), jnp.float32)
        gh = jnp.zeros((_B, _LANES), jnp.float32)
        for o in range(_O):
            c0 = o * 10
            sel = boc == o
            lab = jnp.where(sel, boxes_ref[:, c0 + 9:c0 + 10], lab)
            gcx = jnp.where(sel, boxes_ref[:, c0 + 4:c0 + 5], gcx)
            gcy = jnp.where(sel, boxes_ref[:, c0 + 5:c0 + 6], gcy)
            gw = jnp.where(sel, boxes_ref[:, c0 + 6:c0 + 7], gw)
            gh = jnp.where(sel, boxes_ref[:, c0 + 7:c0 + 8], gh)
        labv = jnp.where(bic < _THRESHOLD, 0, lab.astype(jnp.int32))
        labv = jnp.where(pvec < _P, labv, 0)
        label_ref[:, sl] = labv
        posf = (labv != 0).astype(jnp.float32)
        t0 = (gcx - priors_ref[4, c]) * priors_ref[6, c]
        t1 = (gcy - priors_ref[5, c]) * priors_ref[7, c]
        t2 = jnp.log(gw * priors_ref[8, c]) * 5.0
        t3 = jnp.log(gh * priors_ref[9, c]) * 5.0
        return labs_acc + (jnp.abs(locs_ref[0, :, sl] - t0) +
                           jnp.abs(locs_ref[1, :, sl] - t1) +
                           jnp.abs(locs_ref[2, :, sl] - t2) +
                           jnp.abs(locs_ref[3, :, sl] - t3)) * posf

    labs_ref[...] = lax.fori_loop(
        0, _ROWS, p2_body, jnp.zeros((_B, _LANES), jnp.float32))


def _ce_body(scores_ref, lbl_ref, ce_ref):
    st = jnp.transpose(scores_ref[0])
    lblr = lbl_ref[0]
    e = jnp.exp(st)
    den = jnp.sum(e, axis=0, keepdims=True)
    onehot = lax.broadcasted_iota(jnp.int32, (_C, _P), 0) == lblr
    strue = jnp.sum(jnp.where(onehot, st, 0.0), axis=0, keepdims=True)
    ce_ref[0] = jnp.log(den) - strue


def _mine_body(ce_ref, label_ref, labs_ref, out_ref):
    lab = label_ref[...]
    pos = lab != 0
    posf = pos.astype(jnp.float32)
    ce = ce_ref[...]
    ce_neg = jnp.where(pos, 0.0, ce)
    ceb = lax.bitcast_convert_type(ce_neg, jnp.int32)

    npos = jnp.sum(posf, axis=1, keepdims=True)
    cpos = jnp.sum(ce * posf)
    k = (npos * _NEG_POS_RATIO).astype(jnp.int32)

    lo = jnp.zeros_like(k)
    for bit in range(30, -1, -1):
        cand = lo | jnp.int32(1 << bit)
        cnt = jnp.sum((ceb >= cand).astype(jnp.int32), axis=1, keepdims=True)
        lo = jnp.where(cnt >= k, cand, lo)

    gt = ceb > lo
    hard_sum = jnp.sum(jnp.where(gt, ce_neg, 0.0), axis=1, keepdims=True)
    cnt_gt = jnp.sum(gt.astype(jnp.int32), axis=1, keepdims=True)
    t = lax.bitcast_convert_type(lo, jnp.float32)
    hard_row = jnp.where(k > 0,
                         hard_sum + t * (k - cnt_gt).astype(jnp.float32),
                         0.0)
    hard = jnp.sum(hard_row)

    labs = jnp.sum(labs_ref[...])

    npt = jnp.sum(npos)
    loss = (hard + cpos) / npt + _ALPHA * labs / (npt * 4.0)
    out_ref[...] = jnp.full((1, _LANES), loss, jnp.float32)


@jax.jit
def _run(predicted_locs, predicted_scores, boxes, labels, priors_cxcy):
    B = predicted_scores.shape[0]

    pxy0 = priors_cxcy[:, :2] - priors_cxcy[:, 2:] / 2.0
    pxy1 = priors_cxcy[:, :2] + priors_cxcy[:, 2:] / 2.0
    parea = (pxy1[:, 0] - pxy0[:, 0]) * (pxy1[:, 1] - pxy0[:, 1])
    cols = [pxy0[:, 0], pxy0[:, 1], pxy1[:, 0], pxy1[:, 1],
            priors_cxcy[:, 0], priors_cxcy[:, 1],
            10.0 / priors_cxcy[:, 2], 10.0 / priors_cxcy[:, 3],
            1.0 / priors_cxcy[:, 2], 1.0 / priors_cxcy[:, 3], parea]
    pstack = jnp.stack(cols, 0)
    pad_vals = jnp.array([-5.5, -5.5, -4.5, -4.5, -5.0, -5.0,
                          10.0, 10.0, 1.0, 1.0, 1.0], jnp.float32)
    pad_blk = jnp.broadcast_to(pad_vals[:, None], (11, _PPAD - _P))
    pstack = jnp.concatenate([pstack, pad_blk], 1).reshape(11, _ROWS, _LANES)

    bxy0 = boxes[..., :2]
    bxy1 = boxes[..., 2:]
    bwh = bxy1 - bxy0
    bcxy = (bxy0 + bxy1) / 2.0
    barea = (bwh[..., 0] * bwh[..., 1])[..., None]
    boxes_aug = jnp.concatenate(
        [bxy0, bxy1, bcxy, bwh, barea, labels[..., None].astype(jnp.float32)],
        -1).reshape(B, _O * 10)
    boxes_flat = jnp.pad(boxes_aug, ((0, 0), (0, _LANES - _O * 10)))

    locs_t = jnp.moveaxis(predicted_locs, 2, 0)
    locs_t = jnp.pad(locs_t, ((0, 0), (0, 0), (0, _PPAD - _P)))

    label_rows, labs_part = pl.pallas_call(
        _match_body,
        in_specs=[
            pl.BlockSpec((B, _LANES), lambda: (0, 0)),
            pl.BlockSpec((11, _ROWS, _LANES), lambda: (0, 0, 0)),
            pl.BlockSpec((4, B, _PPAD), lambda: (0, 0, 0)),
        ],
        out_specs=[
            pl.BlockSpec((B, _PPAD), lambda: (0, 0)),
            pl.BlockSpec((B, _LANES), lambda: (0, 0)),
        ],
        out_shape=[
            jax.ShapeDtypeStruct((B, _PPAD), jnp.int32),
            jax.ShapeDtypeStruct((B, _LANES), jnp.float32),
        ],
        scratch_shapes=[
            pltpu.VMEM((B, _PPAD), jnp.float32),
            pltpu.VMEM((B, _PPAD), jnp.int32),
        ],
    )(boxes_flat, pstack, locs_t)

    lbl_rows = label_rows[:, :_P]
    lbl_r3 = lbl_rows.reshape(B, 1, _P)
    ce_r3 = pl.pallas_call(
        _ce_body,
        grid=(B,),
        in_specs=[
            pl.BlockSpec((1, _P, _C), lambda i: (i, 0, 0)),
            pl.BlockSpec((1, 1, _P), lambda i: (i, 0, 0)),
        ],
        out_specs=pl.BlockSpec((1, 1, _P), lambda i: (i, 0, 0)),
        out_shape=jax.ShapeDtypeStruct((B, 1, _P), jnp.float32),
    )(predicted_scores, lbl_r3)

    ce_rows = ce_r3.reshape(B, _P)
    out = pl.pallas_call(
        _mine_body,
        in_specs=[
            pl.BlockSpec((B, _P), lambda: (0, 0)),
            pl.BlockSpec((B, _P), lambda: (0, 0)),
            pl.BlockSpec((B, _LANES), lambda: (0, 0)),
        ],
        out_specs=pl.BlockSpec((1, _LANES), lambda: (0, 0)),
        out_shape=jax.ShapeDtypeStruct((1, _LANES), jnp.float32),
    )(ce_rows, lbl_rows, labs_part)

    return out[0, 0]


def kernel(predicted_locs, predicted_scores, boxes, labels, priors_cxcy):
    return _run(predicted_locs, predicted_scores, boxes, labels, priors_cxcy)


# speedup vs baseline: 19.8933x; 1.1631x over previous
import jax
import jax.numpy as jnp
from jax import lax
from jax.experimental import pallas as pl
from jax.experimental.pallas import tpu as pltpu

_THRESHOLD = 0.5
_NEG_POS_RATIO = 3.0
_ALPHA = 1.0

_P = 8732
_PPAD = 8832
_ROWS = 69
_LANES = 128
_O = 12
_C = 21
_B = 64
_BIG = 2**30


def _match_body(boxes_ref, priors_ref, locs_ref, label_ref, labs_ref,
                bi_ref, bo_ref):
    lane = lax.broadcasted_iota(jnp.int32, (_B, _LANES), 1)

    pp = jnp.zeros((_B, _LANES), jnp.int32)
    for o in range(_O):
        c0 = o * 10
        bx0 = boxes_ref[:, c0:c0 + 1]
        by0 = boxes_ref[:, c0 + 1:c0 + 2]
        bx1 = boxes_ref[:, c0 + 2:c0 + 3]
        by1 = boxes_ref[:, c0 + 3:c0 + 4]
        barea = boxes_ref[:, c0 + 8:c0 + 9]

        def p1_body(c3, carry, o=o, bx0=bx0, by0=by0, bx1=bx1, by1=by1,
                    barea=barea):
            bl, ci = carry
            for j in range(3):
                c = c3 * 3 + j
                sl = pl.ds(c * _LANES, _LANES)
                inter = jnp.maximum(jnp.minimum(priors_ref[2, c], bx1) -
                                    jnp.maximum(priors_ref[0, c], bx0), 0.0) * \
                    jnp.maximum(jnp.minimum(priors_ref[3, c], by1) -
                                jnp.maximum(priors_ref[1, c], by0), 0.0)
                iou = inter / (priors_ref[10, c] + barea - inter)
                upd = iou > bl
                bl = jnp.where(upd, iou, bl)
                ci = jnp.where(upd, c, ci)
                if o == 0:
                    bi_ref[:, sl] = iou
                    bo_ref[:, sl] = jnp.zeros((_B, _LANES), jnp.int32)
                else:
                    bic = bi_ref[:, sl]
                    upd2 = iou > bic
                    bi_ref[:, sl] = jnp.where(upd2, iou, bic)
                    bo_ref[:, sl] = jnp.where(upd2, o, bo_ref[:, sl])
            return bl, ci

        bl, ci = lax.fori_loop(
            0, _ROWS // 3, p1_body,
            (jnp.zeros((_B, _LANES), jnp.float32),
             jnp.zeros((_B, _LANES), jnp.int32)))
        m = jnp.max(bl, axis=1, keepdims=True)
        p_o = jnp.min(jnp.where(bl == m, ci * _LANES + lane, _BIG),
                      axis=1, keepdims=True)
        pp = jnp.where(lane == o, p_o, pp)

    def p2_body(c3, labs_acc):
        for j in range(3):
            c = c3 * 3 + j
            sl = pl.ds(c * _LANES, _LANES)
            pvec = lane + c * _LANES
            bic = bi_ref[:, sl]
            boc = bo_ref[:, sl]
            for o in range(_O):
                hit = pvec == lax.slice(pp, (0, o), (_B, o + 1))
                boc = jnp.where(hit, o, boc)
                bic = jnp.where(hit, 1.0, bic)
            lab = jnp.zeros((_B, _LANES), jnp.float32)
            gcx = jnp.zeros((_B, _LANES), jnp.float32)
            gcy = jnp.zeros((_B, _LANES), jnp.float32)
            gw = jnp.zeros((_B, _LANES), jnp.float32)
            gh = jnp.zeros((_B, _LANES), jnp.float32)
            for o in range(_O):
                c0 = o * 10
                sel = boc == o
                lab = jnp.where(sel, boxes_ref[:, c0 + 9:c0 + 10], lab)
                gcx = jnp.where(sel, boxes_ref[:, c0 + 4:c0 + 5], gcx)
                gcy = jnp.where(sel, boxes_ref[:, c0 + 5:c0 + 6], gcy)
                gw = jnp.where(sel, boxes_ref[:, c0 + 6:c0 + 7], gw)
                gh = jnp.where(sel, boxes_ref[:, c0 + 7:c0 + 8], gh)
            labv = jnp.where(bic < _THRESHOLD, 0, lab.astype(jnp.int32))
            labv = jnp.where(pvec < _P, labv, 0)
            label_ref[:, sl] = labv
            posf = (labv != 0).astype(jnp.float32)
            t0 = (gcx - priors_ref[4, c]) * priors_ref[6, c]
            t1 = (gcy - priors_ref[5, c]) * priors_ref[7, c]
            t2 = jnp.log(gw * priors_ref[8, c]) * 5.0
            t3 = jnp.log(gh * priors_ref[9, c]) * 5.0
            labs_acc = labs_acc + (jnp.abs(locs_ref[0, :, sl] - t0) +
                                   jnp.abs(locs_ref[1, :, sl] - t1) +
                                   jnp.abs(locs_ref[2, :, sl] - t2) +
                                   jnp.abs(locs_ref[3, :, sl] - t3)) * posf
        return labs_acc

    labs_ref[...] = lax.fori_loop(
        0, _ROWS // 3, p2_body, jnp.zeros((_B, _LANES), jnp.float32))


def _ce_body(scores_ref, lbl_ref, ce_ref):
    st = jnp.transpose(scores_ref[0])
    lblr = lbl_ref[0]
    e = jnp.exp(st)
    den = jnp.sum(e, axis=0, keepdims=True)
    onehot = lax.broadcasted_iota(jnp.int32, (_C, _P), 0) == lblr
    strue = jnp.sum(jnp.where(onehot, st, 0.0), axis=0, keepdims=True)
    ce_ref[0] = jnp.log(den) - strue


def _mine_body(ce_ref, label_ref, labs_ref, out_ref):
    lab = label_ref[...]
    pos = lab != 0
    posf = pos.astype(jnp.float32)
    ce = ce_ref[...]
    ce_neg = jnp.where(pos, 0.0, ce)
    ceb = lax.bitcast_convert_type(ce_neg, jnp.int32)

    npos = jnp.sum(posf, axis=1, keepdims=True)
    cpos = jnp.sum(ce * posf)
    k = (npos * _NEG_POS_RATIO).astype(jnp.int32)

    lo = jnp.zeros_like(k)
    for bit in range(30, -1, -1):
        cand = lo | jnp.int32(1 << bit)
        cnt = jnp.sum((ceb >= cand).astype(jnp.int32), axis=1, keepdims=True)
        lo = jnp.where(cnt >= k, cand, lo)

    gt = ceb > lo
    hard_sum = jnp.sum(jnp.where(gt, ce_neg, 0.0), axis=1, keepdims=True)
    cnt_gt = jnp.sum(gt.astype(jnp.int32), axis=1, keepdims=True)
    t = lax.bitcast_convert_type(lo, jnp.float32)
    hard_row = jnp.where(k > 0,
                         hard_sum + t * (k - cnt_gt).astype(jnp.float32),
                         0.0)
    hard = jnp.sum(hard_row)

    labs = jnp.sum(labs_ref[...])

    npt = jnp.sum(npos)
    loss = (hard + cpos) / npt + _ALPHA * labs / (npt * 4.0)
    out_ref[...] = jnp.full((1, _LANES), loss, jnp.float32)


@jax.jit
def _run(predicted_locs, predicted_scores, boxes, labels, priors_cxcy):
    B = predicted_scores.shape[0]

    pxy0 = priors_cxcy[:, :2] - priors_cxcy[:, 2:] / 2.0
    pxy1 = priors_cxcy[:, :2] + priors_cxcy[:, 2:] / 2.0
    parea = (pxy1[:, 0] - pxy0[:, 0]) * (pxy1[:, 1] - pxy0[:, 1])
    cols = [pxy0[:, 0], pxy0[:, 1], pxy1[:, 0], pxy1[:, 1],
            priors_cxcy[:, 0], priors_cxcy[:, 1],
            10.0 / priors_cxcy[:, 2], 10.0 / priors_cxcy[:, 3],
            1.0 / priors_cxcy[:, 2], 1.0 / priors_cxcy[:, 3], parea]
    pstack = jnp.stack(cols, 0)
    pad_vals = jnp.array([-5.5, -5.5, -4.5, -4.5, -5.0, -5.0,
                          10.0, 10.0, 1.0, 1.0, 1.0], jnp.float32)
    pad_blk = jnp.broadcast_to(pad_vals[:, None], (11, _PPAD - _P))
    pstack = jnp.concatenate([pstack, pad_blk], 1).reshape(11, _ROWS, _LANES)

    bxy0 = boxes[..., :2]
    bxy1 = boxes[..., 2:]
    bwh = bxy1 - bxy0
    bcxy = (bxy0 + bxy1) / 2.0
    barea = (bwh[..., 0] * bwh[..., 1])[..., None]
    boxes_aug = jnp.concatenate(
        [bxy0, bxy1, bcxy, bwh, barea, labels[..., None].astype(jnp.float32)],
        -1).reshape(B, _O * 10)
    boxes_flat = jnp.pad(boxes_aug, ((0, 0), (0, _LANES - _O * 10)))

    locs_t = jnp.moveaxis(predicted_locs, 2, 0)
    locs_t = jnp.pad(locs_t, ((0, 0), (0, 0), (0, _PPAD - _P)))

    label_rows, labs_part = pl.pallas_call(
        _match_body,
        in_specs=[
            pl.BlockSpec((B, _LANES), lambda: (0, 0)),
            pl.BlockSpec((11, _ROWS, _LANES), lambda: (0, 0, 0)),
            pl.BlockSpec((4, B, _PPAD), lambda: (0, 0, 0)),
        ],
        out_specs=[
            pl.BlockSpec((B, _PPAD), lambda: (0, 0)),
            pl.BlockSpec((B, _LANES), lambda: (0, 0)),
        ],
        out_shape=[
            jax.ShapeDtypeStruct((B, _PPAD), jnp.int32),
            jax.ShapeDtypeStruct((B, _LANES), jnp.float32),
        ],
        scratch_shapes=[
            pltpu.VMEM((B, _PPAD), jnp.float32),
            pltpu.VMEM((B, _PPAD), jnp.int32),
        ],
    )(boxes_flat, pstack, locs_t)

    lbl_rows = label_rows[:, :_P]
    lbl_r3 = lbl_rows.reshape(B, 1, _P)
    ce_r3 = pl.pallas_call(
        _ce_body,
        grid=(B,),
        in_specs=[
            pl.BlockSpec((1, _P, _C), lambda i: (i, 0, 0)),
            pl.BlockSpec((1, 1, _P), lambda i: (i, 0, 0)),
        ],
        out_specs=pl.BlockSpec((1, 1, _P), lambda i: (i, 0, 0)),
        out_shape=jax.ShapeDtypeStruct((B, 1, _P), jnp.float32),
    )(predicted_scores, lbl_r3)

    ce_rows = ce_r3.reshape(B, _P)
    out = pl.pallas_call(
        _mine_body,
        in_specs=[
            pl.BlockSpec((B, _P), lambda: (0, 0)),
            pl.BlockSpec((B, _P), lambda: (0, 0)),
            pl.BlockSpec((B, _LANES), lambda: (0, 0)),
        ],
        out_specs=pl.BlockSpec((1, _LANES), lambda: (0, 0)),
        out_shape=jax.ShapeDtypeStruct((1, _LANES), jnp.float32),
    )(ce_rows, lbl_rows, labs_part)

    return out[0, 0]


def kernel(predicted_locs, predicted_scores, boxes, labels, priors_cxcy):
    return _run(predicted_locs, predicted_scores, boxes, labels, priors_cxcy)


# speedup vs baseline: 37.8728x; 1.9038x over previous
import jax
import jax.numpy as jnp
from jax import lax
from jax.experimental import pallas as pl
from jax.experimental.pallas import tpu as pltpu

_THRESHOLD = 0.5
_NEG_POS_RATIO = 3.0
_ALPHA = 1.0

_P = 8732
_PPAD = 8832
_ROWS = 69
_LANES = 128
_O = 12
_C = 21
_B = 64
_BIG = 2**30


def _match_body(boxes_ref, priors_ref, locs_ref, label_ref, labs_ref,
                bi_ref, bo_ref):
    lane = lax.broadcasted_iota(jnp.int32, (_B, _LANES), 1)

    pp = jnp.zeros((_B, _LANES), jnp.int32)
    for o in range(_O):
        c0 = o * 10
        bx0 = boxes_ref[:, c0:c0 + 1]
        by0 = boxes_ref[:, c0 + 1:c0 + 2]
        bx1 = boxes_ref[:, c0 + 2:c0 + 3]
        by1 = boxes_ref[:, c0 + 3:c0 + 4]
        barea = boxes_ref[:, c0 + 8:c0 + 9]

        def p1_body(c3, carry, o=o, bx0=bx0, by0=by0, bx1=bx1, by1=by1,
                    barea=barea):
            bl, ci = carry
            for j in range(3):
                c = c3 * 3 + j
                sl = pl.ds(c * _LANES, _LANES)
                inter = jnp.maximum(jnp.minimum(priors_ref[2, c], bx1) -
                                    jnp.maximum(priors_ref[0, c], bx0), 0.0) * \
                    jnp.maximum(jnp.minimum(priors_ref[3, c], by1) -
                                jnp.maximum(priors_ref[1, c], by0), 0.0)
                iou = inter / (priors_ref[10, c] + barea - inter)
                upd = iou > bl
                bl = jnp.where(upd, iou, bl)
                ci = jnp.where(upd, c, ci)
                if o == 0:
                    bi_ref[:, sl] = iou
                    bo_ref[:, sl] = jnp.zeros((_B, _LANES), jnp.int32)
                else:
                    bic = bi_ref[:, sl]
                    upd2 = iou > bic
                    bi_ref[:, sl] = jnp.where(upd2, iou, bic)
                    bo_ref[:, sl] = jnp.where(upd2, o, bo_ref[:, sl])
            return bl, ci

        bl, ci = lax.fori_loop(
            0, _ROWS // 3, p1_body,
            (jnp.zeros((_B, _LANES), jnp.float32),
             jnp.zeros((_B, _LANES), jnp.int32)))
        m = jnp.max(bl, axis=1, keepdims=True)
        p_o = jnp.min(jnp.where(bl == m, ci * _LANES + lane, _BIG),
                      axis=1, keepdims=True)
        pp = jnp.where(lane == o, p_o, pp)

    def p2_body(c3, labs_acc):
        for j in range(3):
            c = c3 * 3 + j
            sl = pl.ds(c * _LANES, _LANES)
            pvec = lane + c * _LANES
            bic = bi_ref[:, sl]
            boc = bo_ref[:, sl]
            for o in range(_O):
                hit = pvec == lax.slice(pp, (0, o), (_B, o + 1))
                boc = jnp.where(hit, o, boc)
                bic = jnp.where(hit, 1.0, bic)
            lab = jnp.zeros((_B, _LANES), jnp.float32)
            gcx = jnp.zeros((_B, _LANES), jnp.float32)
            gcy = jnp.zeros((_B, _LANES), jnp.float32)
            gw = jnp.zeros((_B, _LANES), jnp.float32)
            gh = jnp.zeros((_B, _LANES), jnp.float32)
            for o in range(_O):
                c0 = o * 10
                sel = boc == o
                lab = jnp.where(sel, boxes_ref[:, c0 + 9:c0 + 10], lab)
                gcx = jnp.where(sel, boxes_ref[:, c0 + 4:c0 + 5], gcx)
                gcy = jnp.where(sel, boxes_ref[:, c0 + 5:c0 + 6], gcy)
                gw = jnp.where(sel, boxes_ref[:, c0 + 6:c0 + 7], gw)
                gh = jnp.where(sel, boxes_ref[:, c0 + 7:c0 + 8], gh)
            labv = jnp.where(bic < _THRESHOLD, 0, lab.astype(jnp.int32))
            labv = jnp.where(pvec < _P, labv, 0)
            label_ref[:, sl] = labv
            posf = (labv != 0).astype(jnp.float32)
            t0 = (gcx - priors_ref[4, c]) * priors_ref[6, c]
            t1 = (gcy - priors_ref[5, c]) * priors_ref[7, c]
            t2 = jnp.log(gw * priors_ref[8, c]) * 5.0
            t3 = jnp.log(gh * priors_ref[9, c]) * 5.0
            labs_acc = labs_acc + (jnp.abs(locs_ref[0, :, sl] - t0) +
                                   jnp.abs(locs_ref[1, :, sl] - t1) +
                                   jnp.abs(locs_ref[2, :, sl] - t2) +
                                   jnp.abs(locs_ref[3, :, sl] - t3)) * posf
        return labs_acc

    labs_ref[...] = lax.fori_loop(
        0, _ROWS // 3, p2_body, jnp.zeros((_B, _LANES), jnp.float32))


def _ce_body(scores_ref, lbl_ref, ce_ref):
    st = scores_ref[0]
    lblr = lbl_ref[0]
    e = jnp.exp(st)
    den = jnp.sum(e, axis=0, keepdims=True)
    onehot = lax.broadcasted_iota(jnp.int32, (_C, _P), 0) == lblr
    strue = jnp.sum(jnp.where(onehot, st, 0.0), axis=0, keepdims=True)
    ce_ref[0] = jnp.log(den) - strue


def _mine_body(ce_ref, label_ref, labs_ref, out_ref):
    lab = label_ref[...]
    pos = lab != 0
    posf = pos.astype(jnp.float32)
    ce = ce_ref[...]
    ce_neg = jnp.where(pos, 0.0, ce)
    ceb = lax.bitcast_convert_type(ce_neg, jnp.int32)

    npos = jnp.sum(posf, axis=1, keepdims=True)
    cpos = jnp.sum(ce * posf)
    k = (npos * _NEG_POS_RATIO).astype(jnp.int32)

    lo = jnp.zeros_like(k)
    for bit in range(30, -1, -1):
        cand = lo | jnp.int32(1 << bit)
        cnt = jnp.sum((ceb >= cand).astype(jnp.int32), axis=1, keepdims=True)
        lo = jnp.where(cnt >= k, cand, lo)

    gt = ceb > lo
    hard_sum = jnp.sum(jnp.where(gt, ce_neg, 0.0), axis=1, keepdims=True)
    cnt_gt = jnp.sum(gt.astype(jnp.int32), axis=1, keepdims=True)
    t = lax.bitcast_convert_type(lo, jnp.float32)
    hard_row = jnp.where(k > 0,
                         hard_sum + t * (k - cnt_gt).astype(jnp.float32),
                         0.0)
    hard = jnp.sum(hard_row)

    labs = jnp.sum(labs_ref[...])

    npt = jnp.sum(npos)
    loss = (hard + cpos) / npt + _ALPHA * labs / (npt * 4.0)
    out_ref[...] = jnp.full((1, _LANES), loss, jnp.float32)


@jax.jit
def _run(predicted_locs, predicted_scores, boxes, labels, priors_cxcy):
    B = predicted_scores.shape[0]

    pxy0 = priors_cxcy[:, :2] - priors_cxcy[:, 2:] / 2.0
    pxy1 = priors_cxcy[:, :2] + priors_cxcy[:, 2:] / 2.0
    parea = (pxy1[:, 0] - pxy0[:, 0]) * (pxy1[:, 1] - pxy0[:, 1])
    cols = [pxy0[:, 0], pxy0[:, 1], pxy1[:, 0], pxy1[:, 1],
            priors_cxcy[:, 0], priors_cxcy[:, 1],
            10.0 / priors_cxcy[:, 2], 10.0 / priors_cxcy[:, 3],
            1.0 / priors_cxcy[:, 2], 1.0 / priors_cxcy[:, 3], parea]
    pstack = jnp.stack(cols, 0)
    pad_vals = jnp.array([-5.5, -5.5, -4.5, -4.5, -5.0, -5.0,
                          10.0, 10.0, 1.0, 1.0, 1.0], jnp.float32)
    pad_blk = jnp.broadcast_to(pad_vals[:, None], (11, _PPAD - _P))
    pstack = jnp.concatenate([pstack, pad_blk], 1).reshape(11, _ROWS, _LANES)

    bxy0 = boxes[..., :2]
    bxy1 = boxes[..., 2:]
    bwh = bxy1 - bxy0
    bcxy = (bxy0 + bxy1) / 2.0
    barea = (bwh[..., 0] * bwh[..., 1])[..., None]
    boxes_aug = jnp.concatenate(
        [bxy0, bxy1, bcxy, bwh, barea, labels[..., None].astype(jnp.float32)],
        -1).reshape(B, _O * 10)
    boxes_flat = jnp.pad(boxes_aug, ((0, 0), (0, _LANES - _O * 10)))

    locs_t = jnp.moveaxis(predicted_locs, 2, 0)
    locs_t = jnp.pad(locs_t, ((0, 0), (0, 0), (0, _PPAD - _P)))

    label_rows, labs_part = pl.pallas_call(
        _match_body,
        in_specs=[
            pl.BlockSpec((B, _LANES), lambda: (0, 0)),
            pl.BlockSpec((11, _ROWS, _LANES), lambda: (0, 0, 0)),
            pl.BlockSpec((4, B, _PPAD), lambda: (0, 0, 0)),
        ],
        out_specs=[
            pl.BlockSpec((B, _PPAD), lambda: (0, 0)),
            pl.BlockSpec((B, _LANES), lambda: (0, 0)),
        ],
        out_shape=[
            jax.ShapeDtypeStruct((B, _PPAD), jnp.int32),
            jax.ShapeDtypeStruct((B, _LANES), jnp.float32),
        ],
        scratch_shapes=[
            pltpu.VMEM((B, _PPAD), jnp.float32),
            pltpu.VMEM((B, _PPAD), jnp.int32),
        ],
    )(boxes_flat, pstack, locs_t)

    scores_t = jnp.moveaxis(predicted_scores, 2, 1)
    lbl_rows = label_rows[:, :_P]
    lbl_r3 = lbl_rows.reshape(B, 1, _P)
    ce_r3 = pl.pallas_call(
        _ce_body,
        grid=(B,),
        in_specs=[
            pl.BlockSpec((1, _C, _P), lambda i: (i, 0, 0)),
            pl.BlockSpec((1, 1, _P), lambda i: (i, 0, 0)),
        ],
        out_specs=pl.BlockSpec((1, 1, _P), lambda i: (i, 0, 0)),
        out_shape=jax.ShapeDtypeStruct((B, 1, _P), jnp.float32),
    )(scores_t, lbl_r3)

    ce_rows = ce_r3.reshape(B, _P)
    out = pl.pallas_call(
        _mine_body,
        in_specs=[
            pl.BlockSpec((B, _P), lambda: (0, 0)),
            pl.BlockSpec((B, _P), lambda: (0, 0)),
            pl.BlockSpec((B, _LANES), lambda: (0, 0)),
        ],
        out_specs=pl.BlockSpec((1, _LANES), lambda: (0, 0)),
        out_shape=jax.ShapeDtypeStruct((1, _LANES), jnp.float32),
    )(ce_rows, lbl_rows, labs_part)

    return out[0, 0]


def kernel(predicted_locs, predicted_scores, boxes, labels, priors_cxcy):
    return _run(predicted_locs, predicted_scores, boxes, labels, priors_cxcy)
